# baseline (speedup 1.0000x reference)
import functools

import jax
import jax.numpy as jnp
from jax import lax
from jax.experimental import pallas as pl
from jax.experimental.pallas import tpu as pltpu
from jax.experimental.pallas import tpu_sc as plsc

NUM_SPECIAL = 1000
PAD_IDX = NUM_SPECIAL
VOCAB = NUM_SPECIAL + 1
DIM = 64
BATCH, SEQ = 4096, 200

B = BATCH * SEQ
CHUNK = 128
N_CHUNKS = B // CHUNK
NC, NS = 2, 16
NW = NC * NS
CHUNKS_PER_W = N_CHUNKS // NW
NBUF = 3
L = 16
COLS = DIM // L


def _emb_body(idx_hbm, tbl_hbm, out_hbm, tbl_v, idx_v, rows_v, ssem):
    wid = lax.axis_index("s") * NC + lax.axis_index("c")
    c0 = wid * CHUNKS_PER_W

    pltpu.sync_copy(tbl_hbm, tbl_v)
    pltpu.sync_copy(idx_hbm.at[pl.ds(c0, CHUNKS_PER_W)], idx_v)

    def step(g, _):
        slot = lax.rem(g, NBUF)

        @pl.when(g >= NBUF)
        def _():
            pltpu.make_async_copy(
                rows_v.at[slot],
                out_hbm.at[pl.ds((c0 + g - NBUF) * CHUNK, CHUNK)],
                ssem.at[slot]).wait()

        def group(q, _):
            r0 = q * L
            ivec = idx_v[g, pl.ds(r0, L)] * DIM
            for j in range(L):
                base = ivec[j]
                for c in range(COLS):
                    rows_v[slot, r0 + j, pl.ds(c * L, L)] = (
                        tbl_v[pl.ds(base + c * L, L)])
            return 0

        lax.fori_loop(0, CHUNK // L, group, 0)

        pltpu.async_copy(rows_v.at[slot],
                         out_hbm.at[pl.ds((c0 + g) * CHUNK, CHUNK)],
                         ssem.at[slot])
        return 0

    lax.fori_loop(0, CHUNKS_PER_W, step, 0)

    def drain(g, _):
        slot = lax.rem(g, NBUF)
        pltpu.make_async_copy(
            rows_v.at[slot],
            out_hbm.at[pl.ds((c0 + g) * CHUNK, CHUNK)],
            ssem.at[slot]).wait()
        return 0

    lax.fori_loop(CHUNKS_PER_W - NBUF, CHUNKS_PER_W, drain, 0)


@jax.jit
def _emb_lookup(idx2d, embs_flat):
    mesh = plsc.VectorSubcoreMesh(core_axis_name="c", subcore_axis_name="s")
    f = pl.kernel(
        _emb_body,
        out_type=jax.ShapeDtypeStruct((B, DIM), jnp.float32),
        mesh=mesh,
        scratch_types=[
            pltpu.VMEM((VOCAB * DIM,), jnp.float32),
            pltpu.VMEM((CHUNKS_PER_W, CHUNK), jnp.int32),
            pltpu.VMEM((NBUF, CHUNK, DIM), jnp.float32),
            pltpu.SemaphoreType.DMA((NBUF,)),
        ],
        compiler_params=pltpu.CompilerParams(use_tc_tiling_on_sc=False),
    )
    return f(idx2d, embs_flat)


def kernel(inputs, embs):
    idx2d = inputs.reshape(N_CHUNKS, CHUNK)
    out = _emb_lookup(idx2d, embs.reshape(-1))
    return out.reshape(BATCH, SEQ, DIM)
